# baseline (speedup 1.0000x reference)
import functools

import jax
import jax.numpy as jnp
from jax import lax
from jax.experimental import pallas as pl
from jax.experimental.pallas import tpu as pltpu
from jax.experimental.pallas import tpu_sc as plsc

F_SP = 26
F_DN = 13
D = 16
EPS = 1e-5

NC = 2
NS = 16
NW = NC * NS
CHUNK = 128


def _sc_gather_sum(idx_t, tables):
    n_grp = idx_t.shape[1]
    b = n_grp * CHUNK
    nchunk = n_grp // NW
    mesh = plsc.VectorSubcoreMesh(core_axis_name="c", subcore_axis_name="s")

    @functools.partial(
        pl.kernel,
        out_type=jax.ShapeDtypeStruct((b, D), jnp.float32),
        mesh=mesh,
        scratch_types=[
            pltpu.VMEM((F_SP, 1, CHUNK), jnp.int32),
            pltpu.VMEM((F_SP * CHUNK, D), jnp.float32),
            pltpu.VMEM((CHUNK, D), jnp.float32),
            pltpu.SemaphoreType.DMA,
        ],
        compiler_params=pltpu.CompilerParams(use_tc_tiling_on_sc=False),
    )
    def k(idx_hbm, tab_hbm, out_hbm, idx_v, rows_v, red_v, gsem):
        wid = lax.axis_index("c") * NS + lax.axis_index("s")

        @pl.loop(0, nchunk)
        def _chunk(kc):
            grp = wid * nchunk + kc
            pltpu.sync_copy(idx_hbm.at[:, pl.ds(grp, 1), :], idx_v)
            for f in range(F_SP):
                pltpu.async_copy(
                    tab_hbm.at[f].at[idx_v.at[f, 0]],
                    rows_v.at[pl.ds(f * CHUNK, CHUNK), :],
                    gsem,
                )
            pltpu.make_async_copy(
                tab_hbm.at[0, pl.ds(0, F_SP * CHUNK), :], rows_v, gsem
            ).wait()

            @pl.loop(0, CHUNK, unroll=2)
            def _row(i):
                a0 = rows_v[i]
                a1 = rows_v[CHUNK + i]
                a2 = rows_v[2 * CHUNK + i]
                a3 = rows_v[3 * CHUNK + i]
                for f in range(4, F_SP - 2, 4):
                    a0 = a0 + rows_v[f * CHUNK + i]
                    a1 = a1 + rows_v[(f + 1) * CHUNK + i]
                    a2 = a2 + rows_v[(f + 2) * CHUNK + i]
                    a3 = a3 + rows_v[(f + 3) * CHUNK + i]
                a0 = a0 + rows_v[(F_SP - 2) * CHUNK + i]
                a1 = a1 + rows_v[(F_SP - 1) * CHUNK + i]
                red_v[i] = (a0 + a1) + (a2 + a3)

            pltpu.sync_copy(red_v, out_hbm.at[pl.ds(grp * CHUNK, CHUNK), :])

    return k(idx_t, tables)


def _tc_combine(dense, acc, gamma, beta, wt, bias):
    def body(dense_ref, acc_ref, g_ref, b_ref, w_ref, bias_ref, out_ref):
        d = dense_ref[...]
        mean = jnp.mean(d, axis=0, keepdims=True)
        c = d - mean
        var = jnp.mean(c * c, axis=0, keepdims=True)
        bn = c * lax.rsqrt(var + EPS) * g_ref[...][None, :] + b_ref[...][None, :]
        dense_logit = jnp.sum(bn * w_ref[...], axis=1, keepdims=True)
        sparse_logit = jnp.sum(acc_ref[...], axis=1, keepdims=True)
        out_ref[...] = sparse_logit + dense_logit + bias_ref[...][None, :]

    return pl.pallas_call(
        body,
        out_shape=jax.ShapeDtypeStruct((dense.shape[0], 1), jnp.float32),
    )(dense, acc, gamma, beta, wt, bias)


def kernel(inputs, tables, gamma, beta, W, bias):
    b = inputs.shape[0]
    idx_t = inputs[:, :F_SP].astype(jnp.int32).T.reshape(F_SP, b // CHUNK, CHUNK)
    acc = _sc_gather_sum(idx_t, tables)
    dense = inputs[:, F_SP:]
    wt = W.reshape(1, F_DN)
    return _tc_combine(dense, acc, gamma, beta, wt, bias)
